# Initial kernel scaffold; baseline (speedup 1.0000x reference)
import jax
import jax.numpy as jnp
from jax.experimental import pallas as pl


def kernel(in_feat, edge_index, W1, b1, W2, b2, W3, b3):
    raise NotImplementedError("write your pallas kernel here")


# speedup vs baseline: 4.1941x; 4.1941x over previous
import functools

import jax
import jax.numpy as jnp
from jax import lax
from jax.experimental import pallas as pl
from jax.experimental.pallas import tpu as pltpu
from jax.experimental.pallas import tpu_sc as plsc

_NC = 2
_NS = 16
_LANE = 128


def _edge_block(e_per_s: int) -> int:
    for b in (128, 120, 112, 104, 96, 88, 80, 72, 64, 56, 48, 40, 32, 24, 16, 8):
        if e_per_s % b == 0:
            return b
    return 0


@functools.lru_cache(maxsize=None)
def _seg_accum(C: int, N: int, E: int):
    assert C % _NC == 0 and N % _NS == 0 and E % _NS == 0
    cpc = C // _NC
    n_per_s = N // _NS
    e_per_s = E // _NS
    B = _edge_block(e_per_s)
    assert B > 0
    nblk = e_per_s // B

    mesh = plsc.VectorSubcoreMesh(
        core_axis_name="c", subcore_axis_name="s", num_cores=_NC, num_subcores=_NS
    )

    n_hi = ((N // _NS) + 7) // 8 * 8
    n_lo = N - (_NS - 1) * n_hi
    assert n_lo > 0 and n_lo % 8 == 0

    def _row_copy(src_ref, dst_ref, src_base, dst_base, sub):
        @pl.when(sub < _NS - 1)
        def _():
            s = pl.multiple_of(src_base + sub * n_hi, 8)
            d = pl.multiple_of(dst_base + sub * n_hi, 8)
            pltpu.sync_copy(src_ref.at[pl.ds(s, n_hi)], dst_ref.at[pl.ds(d, n_hi)])

        @pl.when(sub == _NS - 1)
        def _():
            s = pl.multiple_of(src_base + (_NS - 1) * n_hi, 8)
            d = pl.multiple_of(dst_base + (_NS - 1) * n_hi, 8)
            pltpu.sync_copy(src_ref.at[pl.ds(s, n_lo)], dst_ref.at[pl.ds(d, n_lo)])

    @functools.partial(
        pl.kernel,
        out_type=jax.ShapeDtypeStruct((C * N, _LANE), jnp.float32),
        mesh=mesh,
        scratch_types=[
            pltpu.VMEM((nblk, B), jnp.int32),
            pltpu.VMEM((nblk, B), jnp.int32),
            pltpu.VMEM((B, _LANE), jnp.float32),
            pltpu.VMEM_SHARED((N, _LANE), jnp.float32),
            pltpu.SemaphoreType.DMA,
        ],
    )
    def seg_kernel(x_hbm, srco_hbm, dst_hbm, out_hbm, src_v, dst_v, rows_v, acc_sh, sem):
        core = lax.axis_index("c")
        sub = lax.axis_index("s")
        pltpu.sync_copy(dst_hbm.at[sub], dst_v)
        for j in range(cpc):
            c = core * cpc + j
            pltpu.sync_copy(srco_hbm.at[c * _NS + sub], src_v)
            _row_copy(x_hbm, acc_sh, c * N, 0, sub)
            plsc.subcore_barrier()

            @pl.loop(0, nblk)
            def _(i):
                pltpu.async_copy(x_hbm.at[src_v.at[i]], rows_v, sem).wait()
                pltpu.sync_copy(rows_v, acc_sh.at[dst_v.at[i]], add=True)

            plsc.subcore_barrier()
            _row_copy(acc_sh, out_hbm, 0, c * N, sub)

    return seg_kernel


@functools.lru_cache(maxsize=None)
def _gin_linear(C_in: int, C_out: int, N: int, BN: int):
    D_out = C_out * _LANE
    grid = (N // BN,)

    def body(s_ref, w_ref, b_ref, o_ref):
        acc = jnp.dot(s_ref[0], w_ref[0], preferred_element_type=jnp.float32)
        for c in range(1, C_in):
            acc += jnp.dot(s_ref[c], w_ref[c], preferred_element_type=jnp.float32)
        acc = acc + b_ref[...]
        h = jnp.where(acc >= 0, acc, 0.01 * acc)
        for j in range(C_out):
            o_ref[j] = h[:, j * _LANE:(j + 1) * _LANE]

    return pl.pallas_call(
        body,
        grid=grid,
        in_specs=[
            pl.BlockSpec((C_in, BN, _LANE), lambda i: (0, i, 0)),
            pl.BlockSpec((C_in, _LANE, D_out), lambda i: (0, 0, 0)),
            pl.BlockSpec((1, D_out), lambda i: (0, 0)),
        ],
        out_specs=pl.BlockSpec((C_out, BN, _LANE), lambda i: (0, i, 0)),
        out_shape=jax.ShapeDtypeStruct((C_out, N, _LANE), jnp.float32),
    )


@functools.lru_cache(maxsize=None)
def _gin_final(C_in: int, N: int, BN: int, n_classes: int):
    D_h = 512
    grid = (N // BN,)

    def body(s_ref, w2_ref, b2_ref, w3_ref, b3_ref, o_ref, acc_ref):
        i = pl.program_id(0)
        z = jnp.dot(s_ref[0], w2_ref[0], preferred_element_type=jnp.float32)
        for c in range(1, C_in):
            z += jnp.dot(s_ref[c], w2_ref[c], preferred_element_type=jnp.float32)
        z = z + b2_ref[...]
        h = jnp.where(z >= 0, z, 0.01 * z)
        colsum = jnp.sum(h, axis=0, keepdims=True)

        @pl.when(i == 0)
        def _():
            acc_ref[...] = colsum

        @pl.when(i > 0)
        def _():
            acc_ref[...] = acc_ref[...] + colsum

        @pl.when(i == pl.num_programs(0) - 1)
        def _():
            o_ref[...] = (
                jnp.dot(acc_ref[...], w3_ref[...], preferred_element_type=jnp.float32)
                + b3_ref[...]
            )

    return pl.pallas_call(
        body,
        grid=grid,
        in_specs=[
            pl.BlockSpec((C_in, BN, _LANE), lambda i: (0, i, 0)),
            pl.BlockSpec((C_in, _LANE, D_h), lambda i: (0, 0, 0)),
            pl.BlockSpec((1, D_h), lambda i: (0, 0)),
            pl.BlockSpec((D_h, n_classes), lambda i: (0, 0)),
            pl.BlockSpec((1, n_classes), lambda i: (0, 0)),
        ],
        out_specs=pl.BlockSpec((1, n_classes), lambda i: (0, 0)),
        out_shape=jax.ShapeDtypeStruct((1, n_classes), jnp.float32),
        scratch_shapes=[pltpu.VMEM((1, D_h), jnp.float32)],
    )


def kernel(in_feat, edge_index, W1, b1, W2, b2, W3, b3):
    N, D_in = in_feat.shape
    E = edge_index.shape[1]
    D_h = W1.shape[1]
    n_classes = W3.shape[1]
    C1 = D_in // _LANE
    C2 = D_h // _LANE

    src = edge_index[0].astype(jnp.int32)
    dst = edge_index[1].astype(jnp.int32)

    e_per_s = E // _NS
    B = _edge_block(e_per_s)
    nblk = e_per_s // B

    offs1 = (jnp.arange(C1, dtype=jnp.int32) * N)[:, None]
    offs2 = (jnp.arange(C2, dtype=jnp.int32) * N)[:, None]
    src_o1 = (src[None, :] + offs1).reshape(C1 * _NS, nblk, B)
    src_o2 = (src[None, :] + offs2).reshape(C2 * _NS, nblk, B)
    dst3 = dst.reshape(_NS, nblk, B)

    xc = jnp.transpose(in_feat.reshape(N, C1, _LANE), (1, 0, 2)).reshape(C1 * N, _LANE)

    s1 = _seg_accum(C1, N, E)(xc, src_o1, dst3)
    h1 = _gin_linear(C1, C2, N, 2000)(
        s1.reshape(C1, N, _LANE),
        W1.reshape(C1, _LANE, D_h),
        b1.reshape(1, D_h),
    )
    s2 = _seg_accum(C2, N, E)(h1.reshape(C2 * N, _LANE), src_o2, dst3)
    out = _gin_final(C2, N, 2000, n_classes)(
        s2.reshape(C2, N, _LANE),
        W2.reshape(C2, _LANE, D_h),
        b2.reshape(1, D_h),
        W3,
        b3.reshape(1, n_classes),
    )
    return out
